# baseline (speedup 1.0000x reference)
import functools

import jax
import jax.numpy as jnp
from jax import lax
from jax.experimental import pallas as pl
from jax.experimental.pallas import tpu as pltpu
from jax.experimental.pallas import tpu_sc as plsc

B = 16384
F = 26
FC = 26
D = 32
N = B * F
OUT_ROWS = B * (F + FC)

_info = plsc.get_sparse_core_info()
NC = _info.num_cores
NS = _info.num_subcores
NW = NC * NS
PER_W = N // NW
BATCH = 128
NBATCH = PER_W // BATCH

assert N % NW == 0 and PER_W % BATCH == 0


def _sc_body(idx_hbm, dsta_hbm, dstb_hbm, table_hbm, c_hbm, out_hbm,
             idx_v, dsta_v, dstb_v, rows_a, rows_b, sem_g, sem_c, sem_s):
    wid = lax.axis_index("s") * NC + lax.axis_index("c")
    base = wid * PER_W
    pltpu.sync_copy(idx_hbm.at[wid], idx_v)
    pltpu.sync_copy(dsta_hbm.at[wid], dsta_v)
    pltpu.sync_copy(dstb_hbm.at[wid], dstb_v)

    def step(j, carry):
        g = pltpu.async_copy(table_hbm.at[idx_v.at[j]], rows_a, sem_g)
        c = pltpu.async_copy(c_hbm.at[pl.ds(base + j * BATCH, BATCH)], rows_b, sem_c)
        g.wait()
        c.wait()
        sa = pltpu.async_copy(rows_a, out_hbm.at[dsta_v.at[j]], sem_s)
        sb = pltpu.async_copy(rows_b, out_hbm.at[dstb_v.at[j]], sem_s)
        sa.wait()
        sb.wait()
        return carry

    lax.fori_loop(0, NBATCH, step, 0)


_sc_call = functools.partial(
    pl.kernel,
    mesh=plsc.VectorSubcoreMesh(core_axis_name="c", subcore_axis_name="s"),
    compiler_params=pltpu.CompilerParams(use_tc_tiling_on_sc=False),
    out_type=jax.ShapeDtypeStruct((OUT_ROWS, D), jnp.float32),
    scratch_types=[
        pltpu.VMEM((NBATCH, BATCH), jnp.int32),
        pltpu.VMEM((NBATCH, BATCH), jnp.int32),
        pltpu.VMEM((NBATCH, BATCH), jnp.int32),
        pltpu.VMEM((BATCH, D), jnp.float32),
        pltpu.VMEM((BATCH, D), jnp.float32),
        pltpu.SemaphoreType.DMA,
        pltpu.SemaphoreType.DMA,
        pltpu.SemaphoreType.DMA,
    ],
)(_sc_body)


_SB = 512
_RB = _SB * FC


def _cont_body(cont_ref, lowt_ref, hight_ref, o_ref):
    o_ref[...] = lowt_ref[...] + hight_ref[...] * cont_ref[...]


def _cont_tc(cont, low, high):
    cont2 = cont.reshape(N, 1)
    lowt = jnp.tile(low, (_SB, 1))
    hight = jnp.tile(high, (_SB, 1))
    return pl.pallas_call(
        _cont_body,
        grid=(B // _SB,),
        in_specs=[
            pl.BlockSpec((_RB, 1), lambda i: (i, 0)),
            pl.BlockSpec((_RB, D), lambda i: (0, 0)),
            pl.BlockSpec((_RB, D), lambda i: (0, 0)),
        ],
        out_specs=pl.BlockSpec((_RB, D), lambda i: (i, 0)),
        out_shape=jax.ShapeDtypeStruct((N, D), jnp.float32),
    )(cont2, lowt, hight)


def kernel(x, cont, embeds, cont_embed_low, cont_embed_high):
    idx = x.reshape(N).astype(jnp.int32)
    i = jnp.arange(N, dtype=jnp.int32)
    smp = i // F
    fld = i % F
    dsta = smp * (F + FC) + fld
    dstb = dsta + F
    idx3 = idx.reshape(NW, NBATCH, BATCH)
    dsta3 = dsta.reshape(NW, NBATCH, BATCH)
    dstb3 = dstb.reshape(NW, NBATCH, BATCH)
    c_arr = _cont_tc(cont, cont_embed_low, cont_embed_high)
    out2d = _sc_call(idx3, dsta3, dstb3, embeds, c_arr)
    return out2d.reshape(B, F + FC, D)


# speedup vs baseline: 1.1934x; 1.1934x over previous
import functools

import jax
import jax.numpy as jnp
from jax import lax
from jax.experimental import pallas as pl
from jax.experimental.pallas import tpu as pltpu
from jax.experimental.pallas import tpu_sc as plsc

B = 16384
F = 26
FC = 26
D = 32
N = B * F
OUT_ROWS = B * (F + FC)

_info = plsc.get_sparse_core_info()
NC = _info.num_cores
NS = _info.num_subcores
L = _info.num_lanes
NW = NC * NS
PER_W = N // NW
BATCH = 104
NBATCH = PER_W // BATCH

assert N % NW == 0 and PER_W % BATCH == 0 and PER_W % F == 0

_GDN = lax.GatherDimensionNumbers(
    offset_dims=(), collapsed_slice_dims=(0,), start_index_map=(0,))


def _lane_bcast(v, k):
    idx = jnp.full((L, 1), k, jnp.int32)
    return lax.gather(v, idx, _GDN, (1,),
                      mode=lax.GatherScatterMode.PROMISE_IN_BOUNDS)


def _sc_body(idx_hbm, dsta_hbm, dstb_hbm, cont_hbm, low_hbm, high_hbm,
             table_hbm, out_hbm,
             idx_v, dsta_v, dstb_v, cont_v, low_v, high_v, rows_a, rows_b,
             sem_g, sem_s):
    wid = lax.axis_index("s") * NC + lax.axis_index("c")
    pltpu.sync_copy(idx_hbm.at[wid], idx_v)
    pltpu.sync_copy(dsta_hbm.at[wid], dsta_v)
    pltpu.sync_copy(dstb_hbm.at[wid], dstb_v)
    pltpu.sync_copy(cont_hbm.at[wid], cont_v.at[pl.ds(0, PER_W)])
    pltpu.sync_copy(low_hbm, low_v)
    pltpu.sync_copy(high_hbm, high_v)

    def step(j, carry):
        g = pltpu.async_copy(table_hbm.at[idx_v.at[j]], rows_a, sem_g)
        jb = j * BATCH
        for g16 in range((BATCH + L - 1) // L):
            cvv = cont_v[pl.ds(jb + g16 * L, L)]
            for k in range(L):
                r = g16 * L + k
                if r >= BATCH:
                    break
                fo = (r % F) * D
                lo0 = low_v[pl.ds(fo, L)]
                lo1 = low_v[pl.ds(fo + L, L)]
                hi0 = high_v[pl.ds(fo, L)]
                hi1 = high_v[pl.ds(fo + L, L)]
                cv = _lane_bcast(cvv, k)
                rows_b[r, pl.ds(0, L)] = lo0 + hi0 * cv
                rows_b[r, pl.ds(L, L)] = lo1 + hi1 * cv
        g.wait()
        sa = pltpu.async_copy(rows_a, out_hbm.at[dsta_v.at[j]], sem_s)
        sb = pltpu.async_copy(rows_b, out_hbm.at[dstb_v.at[j]], sem_s)
        sa.wait()
        sb.wait()
        return carry

    lax.fori_loop(0, NBATCH, step, 0)


_sc_call = functools.partial(
    pl.kernel,
    mesh=plsc.VectorSubcoreMesh(core_axis_name="c", subcore_axis_name="s"),
    compiler_params=pltpu.CompilerParams(use_tc_tiling_on_sc=False),
    out_type=jax.ShapeDtypeStruct((OUT_ROWS, D), jnp.float32),
    scratch_types=[
        pltpu.VMEM((NBATCH, BATCH), jnp.int32),
        pltpu.VMEM((NBATCH, BATCH), jnp.int32),
        pltpu.VMEM((NBATCH, BATCH), jnp.int32),
        pltpu.VMEM((PER_W + L,), jnp.float32),
        pltpu.VMEM((FC * D,), jnp.float32),
        pltpu.VMEM((FC * D,), jnp.float32),
        pltpu.VMEM((BATCH, D), jnp.float32),
        pltpu.VMEM((BATCH, D), jnp.float32),
        pltpu.SemaphoreType.DMA,
        pltpu.SemaphoreType.DMA,
    ],
)(_sc_body)


def kernel(x, cont, embeds, cont_embed_low, cont_embed_high):
    idx = x.reshape(N).astype(jnp.int32)
    i = jnp.arange(N, dtype=jnp.int32)
    smp = i // F
    fld = i % F
    dsta = smp * (F + FC) + fld
    dstb = dsta + F
    idx3 = idx.reshape(NW, NBATCH, BATCH)
    dsta3 = dsta.reshape(NW, NBATCH, BATCH)
    dstb3 = dstb.reshape(NW, NBATCH, BATCH)
    cont2 = cont.reshape(NW, PER_W)
    low1 = cont_embed_low.reshape(FC * D)
    high1 = cont_embed_high.reshape(FC * D)
    out2d = _sc_call(idx3, dsta3, dstb3, cont2, low1, high1, embeds)
    return out2d.reshape(B, F + FC, D)


# speedup vs baseline: 1.8899x; 1.5837x over previous
import functools

import jax
import jax.numpy as jnp
from jax import lax
from jax.experimental import pallas as pl
from jax.experimental.pallas import tpu as pltpu
from jax.experimental.pallas import tpu_sc as plsc

B = 16384
F = 26
FC = 26
D = 32
N = B * F
W128 = 128

_info = plsc.get_sparse_core_info()
NC = _info.num_cores
NS = _info.num_subcores
NW = NC * NS
PER_W = N // NW
BATCH = 128
NBATCH = PER_W // BATCH

assert N % NW == 0 and PER_W % BATCH == 0


def _sc_body(idx_hbm, table_hbm, emb_hbm, idx_v, rows_v, sem_g):
    wid = lax.axis_index("s") * NC + lax.axis_index("c")
    base = wid * PER_W
    pltpu.sync_copy(idx_hbm.at[wid], idx_v)

    def step(j, carry):
        pltpu.async_copy(table_hbm.at[idx_v.at[j]], rows_v, sem_g).wait()
        pltpu.sync_copy(
            rows_v,
            emb_hbm.at[pl.ds(base + j * BATCH, BATCH), pl.ds(0, D)])
        return carry

    lax.fori_loop(0, NBATCH, step, 0)


_sc_call = functools.partial(
    pl.kernel,
    mesh=plsc.VectorSubcoreMesh(core_axis_name="c", subcore_axis_name="s"),
    compiler_params=pltpu.CompilerParams(use_tc_tiling_on_sc=False),
    out_type=jax.ShapeDtypeStruct((N, W128), jnp.float32),
    scratch_types=[
        pltpu.VMEM((NBATCH, BATCH), jnp.int32),
        pltpu.VMEM((BATCH, D), jnp.float32),
        pltpu.SemaphoreType.DMA,
    ],
)(_sc_body)


_SB = 256


def _tc_body(emb_ref, cont_ref, low_ref, high_ref, o_ref):
    e3 = emb_ref[...][:, :D].reshape(_SB, F, D)
    c = (low_ref[...][None, :, :]
         + high_ref[...][None, :, :] * cont_ref[...][:, :, None])
    o_ref[...] = jnp.concatenate([e3, c], axis=1)


def _tc_call(emb, cont, low, high):
    return pl.pallas_call(
        _tc_body,
        grid=(B // _SB,),
        in_specs=[
            pl.BlockSpec((_SB * F, W128), lambda i: (i, 0)),
            pl.BlockSpec((_SB, F), lambda i: (i, 0)),
            pl.BlockSpec((F, D), lambda i: (0, 0)),
            pl.BlockSpec((F, D), lambda i: (0, 0)),
        ],
        out_specs=pl.BlockSpec((_SB, F + FC, D), lambda i: (i, 0, 0)),
        out_shape=jax.ShapeDtypeStruct((B, F + FC, D), jnp.float32),
    )(emb, cont, low, high)


def kernel(x, cont, embeds, cont_embed_low, cont_embed_high):
    idx3 = x.reshape(N).astype(jnp.int32).reshape(NW, NBATCH, BATCH)
    emb = _sc_call(idx3, embeds)
    return _tc_call(emb, cont, cont_embed_low, cont_embed_high)
